# Initial kernel scaffold; baseline (speedup 1.0000x reference)
import jax
import jax.numpy as jnp
from jax.experimental import pallas as pl


def kernel(x, token_table, pos_table, seg_table):
    raise NotImplementedError("write your pallas kernel here")


# speedup vs baseline: 2.1121x; 2.1121x over previous
import functools

import jax
import jax.numpy as jnp
from jax import lax
from jax.experimental import pallas as pl
from jax.experimental.pallas import tpu as pltpu
from jax.experimental.pallas import tpu_sc as plsc

B, S, E = 4096, 200, 64
NW = 32
CH = 100
NCH = (B * S) // (NW * CH)

_mesh = plsc.VectorSubcoreMesh(core_axis_name="c", subcore_axis_name="s")


@functools.partial(
    pl.kernel,
    out_type=jax.ShapeDtypeStruct((NW * NCH, CH, E), jnp.float32),
    mesh=_mesh,
    scratch_types=[
        pltpu.VMEM((NCH, CH), jnp.int32),
        pltpu.VMEM((S, E), jnp.float32),
        pltpu.VMEM((E,), jnp.float32),
        pltpu.VMEM((CH, E), jnp.float32),
        pltpu.SemaphoreType.DMA,
    ],
    compiler_params=pltpu.CompilerParams(use_tc_tiling_on_sc=False),
)
def _emb_kernel(x_hbm, tok_hbm, pos_hbm, seg_hbm, out_hbm,
                idx_v, bias_v, seg_v, gbuf, gsem):
    wid = lax.axis_index("s") * 2 + lax.axis_index("c")

    pltpu.sync_copy(x_hbm.at[wid], idx_v)
    pltpu.sync_copy(pos_hbm, bias_v)
    pltpu.sync_copy(seg_hbm.at[1], seg_v)

    def seg_body(r, carry):
        for u in range(E // 16):
            sl = pl.ds(u * 16, 16)
            bias_v[r, sl] = bias_v[r, sl] + seg_v[sl]
        return carry
    lax.fori_loop(0, S, seg_body, 0)

    def chunk_body(j, carry):
        pltpu.async_copy(tok_hbm.at[idx_v.at[j]], gbuf, gsem).wait()
        srow = lax.rem(j, 2) * CH

        def add_body(r, c):
            for u in range(E // 16):
                sl = pl.ds(u * 16, 16)
                gbuf[r, sl] = gbuf[r, sl] + bias_v[srow + r, sl]
            return c
        lax.fori_loop(0, CH, add_body, 0)

        pltpu.sync_copy(gbuf, out_hbm.at[wid * NCH + j])
        return carry
    lax.fori_loop(0, NCH, chunk_body, 0)


def kernel(x, token_table, pos_table, seg_table):
    xi = x.astype(jnp.int32).reshape(NW, NCH, CH)
    out = _emb_kernel(xi, token_table, pos_table, seg_table)
    return out.reshape(B, S, E)


# speedup vs baseline: 4.2230x; 1.9995x over previous
import functools

import jax
import jax.numpy as jnp
from jax import lax
from jax.experimental import pallas as pl
from jax.experimental.pallas import tpu as pltpu
from jax.experimental.pallas import tpu_sc as plsc

B, S, E = 4096, 200, 64
NW = 32
CH = 100
NCH = (B * S) // (NW * CH)
NBUF = 4
NG = NCH // NBUF

_mesh = plsc.VectorSubcoreMesh(core_axis_name="c", subcore_axis_name="s")


@functools.partial(
    pl.kernel,
    out_type=jax.ShapeDtypeStruct((NW * NCH, CH, E), jnp.float32),
    mesh=_mesh,
    scratch_types=(
        [pltpu.VMEM((NCH, CH), jnp.int32),
         pltpu.VMEM((S, E), jnp.float32),
         pltpu.VMEM((E,), jnp.float32)]
        + [pltpu.VMEM((CH, E), jnp.float32)] * NBUF
        + [pltpu.VMEM((CH, E), jnp.float32)] * NBUF
        + [pltpu.SemaphoreType.DMA] * NBUF
        + [pltpu.SemaphoreType.DMA] * NBUF
    ),
    compiler_params=pltpu.CompilerParams(use_tc_tiling_on_sc=False),
)
def _emb_kernel(x_hbm, tok_hbm, pos_hbm, seg_hbm, out_hbm,
                idx_v, bias_v, seg_v, *rest):
    gbufs = rest[0:NBUF]
    wbufs = rest[NBUF:2 * NBUF]
    gsems = rest[2 * NBUF:3 * NBUF]
    osems = rest[3 * NBUF:4 * NBUF]

    wid = lax.axis_index("s") * 2 + lax.axis_index("c")
    obase = wid * NCH

    pltpu.sync_copy(x_hbm.at[wid], idx_v)
    pltpu.sync_copy(pos_hbm, bias_v)
    pltpu.sync_copy(seg_hbm.at[1], seg_v)

    def seg_body(r, carry):
        for u in range(E // 16):
            sl = pl.ds(u * 16, 16)
            bias_v[r, sl] = bias_v[r, sl] + seg_v[sl]
        return carry
    lax.fori_loop(0, S, seg_body, 0)

    def start_gather(j, b):
        pltpu.async_copy(tok_hbm.at[idx_v.at[j]], gbufs[b], gsems[b])

    def wait_gather(j, b):
        pltpu.make_async_copy(tok_hbm.at[idx_v.at[j]], gbufs[b],
                              gsems[b]).wait()

    def start_write(j, b):
        pltpu.async_copy(wbufs[b], out_hbm.at[obase + j], osems[b])

    def wait_write(j, b):
        pltpu.make_async_copy(wbufs[b], out_hbm.at[obase + j],
                              osems[b]).wait()

    for b in range(NBUF):
        start_gather(b, b)

    def round_body(g, carry):
        for b in range(NBUF):
            j = g * NBUF + b
            wait_gather(j, b)

            @pl.when(g > 0)
            def _():
                wait_write(j - NBUF, b)

            srow = (b % 2) * CH

            def add_body(r, c, _b=b, _srow=srow):
                for u in range(E // 16):
                    sl = pl.ds(u * 16, 16)
                    wbufs[_b][r, sl] = gbufs[_b][r, sl] + bias_v[_srow + r, sl]
                return c
            lax.fori_loop(0, CH, add_body, 0)

            start_write(j, b)

            @pl.when(g < NG - 1)
            def _():
                start_gather(j + NBUF, b)
        return carry
    lax.fori_loop(0, NG, round_body, 0)

    for b in range(NBUF):
        wait_write((NG - 1) * NBUF + b, b)


def kernel(x, token_table, pos_table, seg_table):
    xi = x.astype(jnp.int32).reshape(NW, NCH, CH)
    out = _emb_kernel(xi, token_table, pos_table, seg_table)
    return out.reshape(B, S, E)
